# Initial kernel scaffold; baseline (speedup 1.0000x reference)
import jax
import jax.numpy as jnp
from jax.experimental import pallas as pl


def kernel(x, edge_index, edge_type, W):
    raise NotImplementedError("write your pallas kernel here")


# speedup vs baseline: 14.1444x; 14.1444x over previous
import functools

import jax
import jax.numpy as jnp
from jax import lax
from jax.experimental import pallas as pl
from jax.experimental.pallas import tpu as pltpu
from jax.experimental.pallas import tpu_sc as plsc

N_NODES = 10000
N_REL = 4
D = 128

NC = 2
NS = 16
NW = NC * NS
CHUNK = 128
N_PAD = 10240
ROWS_PER_TILE = N_PAD // NS
ROWS_PER_TILE_LAST = N_NODES - (NS - 1) * ROWS_PER_TILE


def _matmul_body(x_ref, w_ref, o_ref):
    o_ref[...] = lax.dot_general(
        x_ref[...], w_ref[0],
        (((1,), (0,)), ((), ())),
        preferred_element_type=jnp.float32,
        precision=lax.Precision.HIGHEST,
    )


def _pre_sup(x, W):
    return pl.pallas_call(
        _matmul_body,
        grid=(10, N_REL),
        in_specs=[
            pl.BlockSpec((1000, D), lambda i, r: (i, 0)),
            pl.BlockSpec((1, D, D), lambda i, r: (r, 0, 0)),
        ],
        out_specs=pl.BlockSpec((1000, D), lambda i, r: (r * 10 + i, 0)),
        out_shape=jax.ShapeDtypeStruct((N_REL * N_NODES, D), jnp.float32),
    )(x, W)


def _combine_body(p_ref, o_ref):
    o_ref[...] = jnp.maximum(p_ref[0] + p_ref[1], 0.0)


def _combine(partial):
    return pl.pallas_call(
        _combine_body,
        grid=(10,),
        in_specs=[pl.BlockSpec((2, 1000, D), lambda i: (0, i, 0))],
        out_specs=pl.BlockSpec((1000, D), lambda i: (i, 0)),
        out_shape=jax.ShapeDtypeStruct((N_NODES, D), jnp.float32),
    )(partial)


def _sc_aggregate(presup, src, etype, dst, zeros, chunks_per_tile):
    mesh = plsc.VectorSubcoreMesh(core_axis_name="c", subcore_axis_name="s")

    @functools.partial(
        pl.kernel,
        out_type=jax.ShapeDtypeStruct((NC * N_NODES, D), jnp.float32),
        mesh=mesh,
        scratch_types=[
            pltpu.VMEM_SHARED((N_PAD, D), jnp.float32),
            pltpu.VMEM((chunks_per_tile * CHUNK,), jnp.int32),
            pltpu.VMEM((chunks_per_tile * CHUNK,), jnp.int32),
            pltpu.VMEM((chunks_per_tile, CHUNK), jnp.int32),
            pltpu.VMEM((CHUNK,), jnp.int32),
            pltpu.VMEM((CHUNK, D), jnp.float32),
            pltpu.SemaphoreType.DMA,
        ],
    )
    def agg(presup_hbm, src_hbm, type_hbm, dst_hbm, zeros_hbm, out_hbm,
            acc, src_v, type_v, dst_v, gidx_v, rows_v, sem):
        c = lax.axis_index("c")
        s = lax.axis_index("s")
        wid = c * NS + s

        pltpu.sync_copy(src_hbm.at[wid], src_v)
        pltpu.sync_copy(type_hbm.at[wid], type_v)
        pltpu.sync_copy(dst_hbm.at[wid], dst_v)

        pltpu.sync_copy(zeros_hbm, acc.at[pl.ds(s * ROWS_PER_TILE, ROWS_PER_TILE)])
        plsc.subcore_barrier()

        @pl.loop(0, chunks_per_tile)
        def _(j):
            @pl.loop(0, CHUNK // 16)
            def _(k):
                t16 = type_v[pl.ds(j * CHUNK + k * 16, 16)]
                s16 = src_v[pl.ds(j * CHUNK + k * 16, 16)]
                gidx_v[pl.ds(k * 16, 16)] = t16 * N_NODES + s16

            pltpu.async_copy(presup_hbm.at[gidx_v], rows_v, sem).wait()
            pltpu.sync_copy(rows_v, acc.at[dst_v.at[j]], add=True)

        plsc.subcore_barrier()

        row0 = s * ROWS_PER_TILE

        @pl.when(s < NS - 1)
        def _():
            pltpu.sync_copy(acc.at[pl.ds(row0, ROWS_PER_TILE)],
                            out_hbm.at[pl.ds(c * N_NODES + row0, ROWS_PER_TILE)])

        @pl.when(s == NS - 1)
        def _():
            pltpu.sync_copy(acc.at[pl.ds((NS - 1) * ROWS_PER_TILE, ROWS_PER_TILE_LAST)],
                            out_hbm.at[pl.ds(c * N_NODES + (NS - 1) * ROWS_PER_TILE,
                                             ROWS_PER_TILE_LAST)])

    return agg(presup, src, etype, dst, zeros)


def kernel(x, edge_index, edge_type, W):
    n_edges = edge_index.shape[1]
    per_tile = -(-n_edges // (NW * CHUNK)) * CHUNK
    e_pad = NW * per_tile
    chunks_per_tile = per_tile // CHUNK

    src = edge_index[0].astype(jnp.int32)
    dst = edge_index[1].astype(jnp.int32)
    et = edge_type.astype(jnp.int32)

    pad = e_pad - n_edges
    src = jnp.concatenate([src, jnp.zeros((pad,), jnp.int32)])
    et = jnp.concatenate([et, jnp.zeros((pad,), jnp.int32)])
    dst = jnp.concatenate([dst, jnp.full((pad,), N_NODES, jnp.int32)])

    src = src.reshape(NW, per_tile)
    et = et.reshape(NW, per_tile)
    dst = dst.reshape(NW, chunks_per_tile, CHUNK)
    zeros = jnp.zeros((ROWS_PER_TILE, D), jnp.float32)

    presup = _pre_sup(x, W)
    partial = _sc_aggregate(presup, src, et, dst, zeros, chunks_per_tile)
    return _combine(partial.reshape(NC, N_NODES, D))
